# Initial kernel scaffold; baseline (speedup 1.0000x reference)
import jax
import jax.numpy as jnp
from jax.experimental import pallas as pl


def kernel(nfeats, efeats, edge_index, params):
    raise NotImplementedError("write your pallas kernel here")


# speedup vs baseline: 1.0693x; 1.0693x over previous
import jax
import jax.numpy as jnp
from jax.experimental import pallas as pl

ATTN_DIMS = (32, 32)
EDGE_DIMS = (16, 16)
HEADS = (4, 4)
NODE_S2S_ITERS = 3
NODE_S2S_LAYERS = 2
EDGE_S2S_ITERS = 1
EDGE_S2S_LAYERS = 2


def _egat_layer(x, ef, src, dst, p, H, out_n, out_e, n_nodes):
    f_ni = x @ p['Wni']
    f_nj = x @ p['Wnj']
    f_fij = ef @ p['Wfij']
    f = f_ni[src] + f_nj[dst] + f_fij
    f = jax.nn.leaky_relu(f, 0.2).reshape(-1, H, out_e)
    e = (f * p['attn'][None]).sum(-1)
    ee = jnp.exp(e)
    denom = jax.ops.segment_sum(ee, dst, num_segments=n_nodes)
    h = (x @ p['Wnode']).reshape(-1, H, out_n)
    m = h[src] * ee[:, :, None]
    h_out = jax.ops.segment_sum(m, dst, num_segments=n_nodes)
    h_out = h_out / (denom[:, :, None] + 1e-9)
    return h_out, f


def _lstm_forward(x, hs, cs, lstm_ps):
    new_hs, new_cs = [], []
    inp = x
    for lp, h, c in zip(lstm_ps, hs, cs):
        gates = inp @ lp['Wih'].T + h @ lp['Whh'].T + lp['bih'] + lp['bhh']
        i, f, g, o = jnp.split(gates, 4, axis=-1)
        c = jax.nn.sigmoid(f) * c + jax.nn.sigmoid(i) * jnp.tanh(g)
        h = jax.nn.sigmoid(o) * jnp.tanh(c)
        new_hs.append(h)
        new_cs.append(c)
        inp = h
    return inp, new_hs, new_cs


def _set2set(feat, lstm_ps, dim, n_iters, n_layers):
    q_star = jnp.zeros((1, 2 * dim), dtype=feat.dtype)
    hs = [jnp.zeros((1, dim), feat.dtype) for _ in range(n_layers)]
    cs = [jnp.zeros((1, dim), feat.dtype) for _ in range(n_layers)]
    for _ in range(n_iters):
        q, hs, cs = _lstm_forward(q_star, hs, cs, lstm_ps)
        e = feat @ q[0]
        alpha = jax.nn.softmax(e)
        readout = (feat * alpha[:, None]).sum(0, keepdims=True)
        q_star = jnp.concatenate([q, readout], axis=-1)
    return q_star


def _head_kernel(vec_ref, w1_ref, b1_ref, w2_ref, b2_ref,
                 gw1_ref, gb1_ref, gw2_ref, gb2_ref, out_ref):
    x = vec_ref[...]
    dot = lambda a, b: jax.lax.dot(a, b, precision=jax.lax.Precision.HIGHEST)
    h = jax.nn.silu(dot(x, w1_ref[...]) + b1_ref[...])
    h = dot(h, w2_ref[...]) + b2_ref[...]
    g = jax.nn.silu(dot(x, gw1_ref[...]) + gb1_ref[...])
    g = jax.nn.sigmoid(dot(g, gw2_ref[...]) + gb2_ref[...])
    out_ref[...] = h * g


def _gated_mlp_pallas(vec, p):
    return pl.pallas_call(
        _head_kernel,
        out_shape=jax.ShapeDtypeStruct((1, 1), jnp.float32),
    )(vec, p['mlp_W1'], p['mlp_b1'][None], p['mlp_W2'], p['mlp_b2'][None],
      p['gate_W1'], p['gate_b1'][None], p['gate_W2'], p['gate_b2'][None])


def kernel(nfeats, efeats, edge_index, params):
    src, dst = edge_index[0], edge_index[1]
    x = params['emb'][nfeats]
    ef = efeats
    n_nodes = x.shape[0]
    n_edges = ef.shape[0]
    for lp, out_n, out_e, H in zip(params['egat'], ATTN_DIMS, EDGE_DIMS, HEADS):
        h, f = _egat_layer(x, ef, src, dst, lp, H, out_n, out_e, n_nodes)
        x = jax.nn.elu(h.reshape(n_nodes, -1))
        ef = jax.nn.elu(f.reshape(n_edges, -1))
    node_dim = ATTN_DIMS[-1] * HEADS[-1]
    edge_dim = EDGE_DIMS[-1] * HEADS[-1]
    node_vec = _set2set(x, params['node_s2s'], node_dim, NODE_S2S_ITERS, NODE_S2S_LAYERS)
    edge_vec = _set2set(ef, params['edge_s2s'], edge_dim, EDGE_S2S_ITERS, EDGE_S2S_LAYERS)
    vec = jnp.concatenate([node_vec[0], edge_vec[0]], axis=-1)[None]
    return _gated_mlp_pallas(vec, params['out'])[0]


# speedup vs baseline: 1.0898x; 1.0192x over previous
import functools

import jax
import jax.numpy as jnp
from jax import lax
from jax.experimental import pallas as pl
from jax.experimental.pallas import tpu as pltpu
from jax.experimental.pallas import tpu_sc as plsc

ATTN_DIMS = (32, 32)
EDGE_DIMS = (16, 16)
HEADS = (4, 4)
NODE_S2S_ITERS = 3
NODE_S2S_LAYERS = 2
EDGE_S2S_ITERS = 1
EDGE_S2S_LAYERS = 2

_HI = jax.lax.Precision.HIGHEST
_MESH = plsc.VectorSubcoreMesh(core_axis_name="c", subcore_axis_name="s")
_NW = 32
_SC_PARAMS = pltpu.CompilerParams(use_tc_tiling_on_sc=False)


def _hdot(a, b):
    return jax.lax.dot(a, b, precision=_HI)


def _tables_body(emb_ref, wni_ref, wnj_ref, wnode_ref, tni_ref, tnj_ref, th_ref):
    e = emb_ref[...]
    tni_ref[...] = _hdot(e, wni_ref[...])
    tnj_ref[...] = _hdot(e, wnj_ref[...])
    th_ref[...] = _hdot(e, wnode_ref[...])


def _node_tables(emb, wni, wnj, wnode):
    t = emb.shape[0]
    return pl.pallas_call(
        _tables_body,
        out_shape=(
            jax.ShapeDtypeStruct((t, wni.shape[1]), jnp.float32),
            jax.ShapeDtypeStruct((t, wnj.shape[1]), jnp.float32),
            jax.ShapeDtypeStruct((t, wnode.shape[1]), jnp.float32),
        ),
    )(emb, wni, wnj, wnode)


def _node_expand(tni, tnj, th, nfeats):
    n = nfeats.shape[0]
    ch = 400
    nb = n // ch
    nbt = -(-nb // _NW)

    @functools.partial(
        pl.kernel,
        mesh=_MESH,
        out_type=(
            jax.ShapeDtypeStruct((n, 64), jnp.float32),
            jax.ShapeDtypeStruct((n, 64), jnp.float32),
            jax.ShapeDtypeStruct((n, 128), jnp.float32),
        ),
        scratch_types=[
            pltpu.VMEM((ch,), jnp.int32),
            pltpu.VMEM((ch, 64), jnp.float32),
            pltpu.VMEM((ch, 64), jnp.float32),
            pltpu.VMEM((ch, 128), jnp.float32),
        ],
        compiler_params=_SC_PARAMS,
    )
    def k(tni_h, tnj_h, th_h, nf_h, oni_h, onj_h, oh_h, idx_v, b1, b2, b3):
        wid = lax.axis_index("s") * 2 + lax.axis_index("c")
        for i in range(nbt):
            b = i * _NW + wid

            @pl.when(b < nb)
            def _():
                base = b * ch
                pltpu.sync_copy(nf_h.at[pl.ds(base, ch)], idx_v)
                pltpu.sync_copy(tni_h.at[idx_v], b1)
                pltpu.sync_copy(tnj_h.at[idx_v], b2)
                pltpu.sync_copy(th_h.at[idx_v], b3)
                pltpu.sync_copy(b1, oni_h.at[pl.ds(base, ch)])
                pltpu.sync_copy(b2, onj_h.at[pl.ds(base, ch)])
                pltpu.sync_copy(b3, oh_h.at[pl.ds(base, ch)])

    return k(tni, tnj, th, nfeats)


def _edge_gather(fni, fnj, src, dst):
    e = src.shape[0]
    d = fni.shape[1]
    ch = 640
    nb = e // ch
    nbt = -(-nb // _NW)

    @functools.partial(
        pl.kernel,
        mesh=_MESH,
        out_type=(
            jax.ShapeDtypeStruct((e, d), jnp.float32),
            jax.ShapeDtypeStruct((e, d), jnp.float32),
        ),
        scratch_types=[
            pltpu.VMEM((ch,), jnp.int32),
            pltpu.VMEM((ch,), jnp.int32),
            pltpu.VMEM((ch, d), jnp.float32),
            pltpu.VMEM((ch, d), jnp.float32),
        ],
        compiler_params=_SC_PARAMS,
    )
    def k(fni_h, fnj_h, src_h, dst_h, gi_h, gj_h, sidx, didx, ba, bb):
        wid = lax.axis_index("s") * 2 + lax.axis_index("c")
        for i in range(nbt):
            b = i * _NW + wid

            @pl.when(b < nb)
            def _():
                base = b * ch
                pltpu.sync_copy(src_h.at[pl.ds(base, ch)], sidx)
                pltpu.sync_copy(dst_h.at[pl.ds(base, ch)], didx)
                pltpu.sync_copy(fni_h.at[sidx], ba)
                pltpu.sync_copy(fnj_h.at[didx], bb)
                pltpu.sync_copy(ba, gi_h.at[pl.ds(base, ch)])
                pltpu.sync_copy(bb, gj_h.at[pl.ds(base, ch)])

    return k(fni, fnj, src, dst)


def _lstm_forward(x, hs, cs, lstm_ps):
    new_hs, new_cs = [], []
    inp = x
    for lp, h, c in zip(lstm_ps, hs, cs):
        gates = inp @ lp['Wih'].T + h @ lp['Whh'].T + lp['bih'] + lp['bhh']
        i, f, g, o = jnp.split(gates, 4, axis=-1)
        c = jax.nn.sigmoid(f) * c + jax.nn.sigmoid(i) * jnp.tanh(g)
        h = jax.nn.sigmoid(o) * jnp.tanh(c)
        new_hs.append(h)
        new_cs.append(c)
        inp = h
    return inp, new_hs, new_cs


def _set2set(feat, lstm_ps, dim, n_iters, n_layers):
    q_star = jnp.zeros((1, 2 * dim), dtype=feat.dtype)
    hs = [jnp.zeros((1, dim), feat.dtype) for _ in range(n_layers)]
    cs = [jnp.zeros((1, dim), feat.dtype) for _ in range(n_layers)]
    for _ in range(n_iters):
        q, hs, cs = _lstm_forward(q_star, hs, cs, lstm_ps)
        e = feat @ q[0]
        alpha = jax.nn.softmax(e)
        readout = (feat * alpha[:, None]).sum(0, keepdims=True)
        q_star = jnp.concatenate([q, readout], axis=-1)
    return q_star


def _head_kernel(vec_ref, w1_ref, b1_ref, w2_ref, b2_ref,
                 gw1_ref, gb1_ref, gw2_ref, gb2_ref, out_ref):
    x = vec_ref[...]
    h = jax.nn.silu(_hdot(x, w1_ref[...]) + b1_ref[...])
    h = _hdot(h, w2_ref[...]) + b2_ref[...]
    g = jax.nn.silu(_hdot(x, gw1_ref[...]) + gb1_ref[...])
    g = jax.nn.sigmoid(_hdot(g, gw2_ref[...]) + gb2_ref[...])
    out_ref[...] = h * g


def _gated_mlp_pallas(vec, p):
    return pl.pallas_call(
        _head_kernel,
        out_shape=jax.ShapeDtypeStruct((1, 1), jnp.float32),
    )(vec, p['mlp_W1'], p['mlp_b1'][None], p['mlp_W2'], p['mlp_b2'][None],
      p['gate_W1'], p['gate_b1'][None], p['gate_W2'], p['gate_b2'][None])


def kernel(nfeats, efeats, edge_index, params):
    src, dst = edge_index[0], edge_index[1]
    n_nodes = nfeats.shape[0]
    n_edges = efeats.shape[0]
    ef = efeats

    for li, (lp, out_n, out_e, H) in enumerate(
            zip(params['egat'], ATTN_DIMS, EDGE_DIMS, HEADS)):
        if li == 0:
            tni, tnj, th = _node_tables(
                params['emb'], lp['Wni'], lp['Wnj'], lp['Wnode'])
            fni, fnj, h = _node_expand(tni, tnj, th, nfeats)
        else:
            fni = _hdot(x, lp['Wni'])
            fnj = _hdot(x, lp['Wnj'])
            h = _hdot(x, lp['Wnode'])
        gi, gj = _edge_gather(fni, fnj, src, dst)
        f = jax.nn.leaky_relu(gi + gj + ef @ lp['Wfij'], 0.2)
        fr = f.reshape(-1, H, out_e)
        z = (fr * lp['attn'][None]).sum(-1)
        ee = jnp.exp(z)
        denom = jax.ops.segment_sum(ee, dst, num_segments=n_nodes)
        hr = h.reshape(-1, H, out_n)
        m = hr[src] * ee[:, :, None]
        h_out = jax.ops.segment_sum(m, dst, num_segments=n_nodes)
        h_out = h_out / (denom[:, :, None] + 1e-9)
        x = jax.nn.elu(h_out.reshape(n_nodes, -1))
        ef = jax.nn.elu(f.reshape(n_edges, -1))

    node_dim = ATTN_DIMS[-1] * HEADS[-1]
    edge_dim = EDGE_DIMS[-1] * HEADS[-1]
    node_vec = _set2set(x, params['node_s2s'], node_dim, NODE_S2S_ITERS, NODE_S2S_LAYERS)
    edge_vec = _set2set(ef, params['edge_s2s'], edge_dim, EDGE_S2S_ITERS, EDGE_S2S_LAYERS)
    vec = jnp.concatenate([node_vec[0], edge_vec[0]], axis=-1)[None]
    return _gated_mlp_pallas(vec, params['out'])[0]


# speedup vs baseline: 1.1392x; 1.0454x over previous
import functools

import jax
import jax.numpy as jnp
from jax import lax
from jax.experimental import pallas as pl
from jax.experimental.pallas import tpu as pltpu
from jax.experimental.pallas import tpu_sc as plsc

ATTN_DIMS = (32, 32)
EDGE_DIMS = (16, 16)
HEADS = (4, 4)
NODE_S2S_ITERS = 3
NODE_S2S_LAYERS = 2
EDGE_S2S_ITERS = 1
EDGE_S2S_LAYERS = 2

_HI = jax.lax.Precision.HIGHEST
_MESH = plsc.VectorSubcoreMesh(core_axis_name="c", subcore_axis_name="s")
_NW = 32
_SC_PARAMS = pltpu.CompilerParams(use_tc_tiling_on_sc=False)


def _hdot(a, b):
    return jax.lax.dot(a, b, precision=_HI)


def _tables_body(emb_ref, wnij_ref, wnode_ref, tnij_ref, th_ref):
    e = emb_ref[...]
    tnij_ref[...] = _hdot(e, wnij_ref[...])
    th_ref[...] = _hdot(e, wnode_ref[...])


def _node_tables(emb, wnij, wnode):
    t = emb.shape[0]
    return pl.pallas_call(
        _tables_body,
        out_shape=(
            jax.ShapeDtypeStruct((t, wnij.shape[1]), jnp.float32),
            jax.ShapeDtypeStruct((t, wnode.shape[1]), jnp.float32),
        ),
    )(emb, wnij, wnode)


def _node_expand(tnij, th, nfeats):
    n = nfeats.shape[0]
    ch = 400
    nb = n // ch
    nbt = -(-nb // _NW)

    @functools.partial(
        pl.kernel,
        mesh=_MESH,
        out_type=(
            jax.ShapeDtypeStruct((n, 128), jnp.float32),
            jax.ShapeDtypeStruct((n, 128), jnp.float32),
        ),
        scratch_types=[
            pltpu.VMEM((ch,), jnp.int32),
            pltpu.VMEM((ch, 128), jnp.float32),
            pltpu.VMEM((ch, 128), jnp.float32),
        ],
        compiler_params=_SC_PARAMS,
    )
    def k(tnij_h, th_h, nf_h, onij_h, oh_h, idx_v, b1, b2):
        wid = lax.axis_index("s") * 2 + lax.axis_index("c")
        for i in range(nbt):
            b = i * _NW + wid

            @pl.when(b < nb)
            def _():
                base = b * ch
                pltpu.sync_copy(nf_h.at[pl.ds(base, ch)], idx_v)
                pltpu.sync_copy(tnij_h.at[idx_v], b1)
                pltpu.sync_copy(th_h.at[idx_v], b2)
                pltpu.sync_copy(b1, onij_h.at[pl.ds(base, ch)])
                pltpu.sync_copy(b2, oh_h.at[pl.ds(base, ch)])

    return k(tnij, th, nfeats)


def _edge_gather(fnij, h, src, dst):
    e = src.shape[0]
    ch = 320
    nb = e // ch
    nbt = -(-nb // _NW)

    @functools.partial(
        pl.kernel,
        mesh=_MESH,
        out_type=(
            jax.ShapeDtypeStruct((e, 128), jnp.float32),
            jax.ShapeDtypeStruct((e, 128), jnp.float32),
            jax.ShapeDtypeStruct((e, 128), jnp.float32),
        ),
        scratch_types=[
            pltpu.VMEM((ch,), jnp.int32),
            pltpu.VMEM((ch,), jnp.int32),
            pltpu.VMEM((ch, 128), jnp.float32),
            pltpu.VMEM((ch, 128), jnp.float32),
            pltpu.VMEM((ch, 128), jnp.float32),
        ],
        compiler_params=_SC_PARAMS,
    )
    def k(fnij_h, h_h, src_h, dst_h, gs_h, gd_h, hs_h, sidx, didx, ba, bb, bc):
        wid = lax.axis_index("s") * 2 + lax.axis_index("c")
        for i in range(nbt):
            b = i * _NW + wid

            @pl.when(b < nb)
            def _():
                base = b * ch
                pltpu.sync_copy(src_h.at[pl.ds(base, ch)], sidx)
                pltpu.sync_copy(dst_h.at[pl.ds(base, ch)], didx)
                pltpu.sync_copy(fnij_h.at[sidx], ba)
                pltpu.sync_copy(fnij_h.at[didx], bb)
                pltpu.sync_copy(h_h.at[sidx], bc)
                pltpu.sync_copy(ba, gs_h.at[pl.ds(base, ch)])
                pltpu.sync_copy(bb, gd_h.at[pl.ds(base, ch)])
                pltpu.sync_copy(bc, hs_h.at[pl.ds(base, ch)])

    return k(fnij, h, src, dst)


def _lstm_forward(x, hs, cs, lstm_ps):
    new_hs, new_cs = [], []
    inp = x
    for lp, h, c in zip(lstm_ps, hs, cs):
        gates = inp @ lp['Wih'].T + h @ lp['Whh'].T + lp['bih'] + lp['bhh']
        i, f, g, o = jnp.split(gates, 4, axis=-1)
        c = jax.nn.sigmoid(f) * c + jax.nn.sigmoid(i) * jnp.tanh(g)
        h = jax.nn.sigmoid(o) * jnp.tanh(c)
        new_hs.append(h)
        new_cs.append(c)
        inp = h
    return inp, new_hs, new_cs


def _set2set(feat, lstm_ps, dim, n_iters, n_layers):
    q_star = jnp.zeros((1, 2 * dim), dtype=feat.dtype)
    hs = [jnp.zeros((1, dim), feat.dtype) for _ in range(n_layers)]
    cs = [jnp.zeros((1, dim), feat.dtype) for _ in range(n_layers)]
    for _ in range(n_iters):
        q, hs, cs = _lstm_forward(q_star, hs, cs, lstm_ps)
        e = feat @ q[0]
        alpha = jax.nn.softmax(e)
        readout = (feat * alpha[:, None]).sum(0, keepdims=True)
        q_star = jnp.concatenate([q, readout], axis=-1)
    return q_star


def _head_kernel(vec_ref, w1_ref, b1_ref, w2_ref, b2_ref,
                 gw1_ref, gb1_ref, gw2_ref, gb2_ref, out_ref):
    x = vec_ref[...]
    h = jax.nn.silu(_hdot(x, w1_ref[...]) + b1_ref[...])
    h = _hdot(h, w2_ref[...]) + b2_ref[...]
    g = jax.nn.silu(_hdot(x, gw1_ref[...]) + gb1_ref[...])
    g = jax.nn.sigmoid(_hdot(g, gw2_ref[...]) + gb2_ref[...])
    out_ref[...] = h * g


def _gated_mlp_pallas(vec, p):
    return pl.pallas_call(
        _head_kernel,
        out_shape=jax.ShapeDtypeStruct((1, 1), jnp.float32),
    )(vec, p['mlp_W1'], p['mlp_b1'][None], p['mlp_W2'], p['mlp_b2'][None],
      p['gate_W1'], p['gate_b1'][None], p['gate_W2'], p['gate_b2'][None])


def kernel(nfeats, efeats, edge_index, params):
    src, dst = edge_index[0], edge_index[1]
    n_nodes = nfeats.shape[0]
    n_edges = efeats.shape[0]
    ef = efeats

    for li, (lp, out_n, out_e, H) in enumerate(
            zip(params['egat'], ATTN_DIMS, EDGE_DIMS, HEADS)):
        wnij = jnp.concatenate([lp['Wni'], lp['Wnj']], axis=1)
        if li == 0:
            tnij, th = _node_tables(params['emb'], wnij, lp['Wnode'])
            fnij, h = _node_expand(tnij, th, nfeats)
        else:
            fnij = _hdot(x, wnij)
            h = _hdot(x, lp['Wnode'])
        fsrc, fdst, hsrc = _edge_gather(fnij, h, src, dst)
        g = fsrc[:, :H * out_e] + fdst[:, H * out_e:]
        f = jax.nn.leaky_relu(g + ef @ lp['Wfij'], 0.2)
        fr = f.reshape(-1, H, out_e)
        z = (fr * lp['attn'][None]).sum(-1)
        ee = jnp.exp(z)
        denom = jax.ops.segment_sum(ee, dst, num_segments=n_nodes)
        hr = hsrc.reshape(-1, H, out_n)
        m = hr * ee[:, :, None]
        h_out = jax.ops.segment_sum(m, dst, num_segments=n_nodes)
        h_out = h_out / (denom[:, :, None] + 1e-9)
        x = jax.nn.elu(h_out.reshape(n_nodes, -1))
        ef = jax.nn.elu(f.reshape(n_edges, -1))

    node_dim = ATTN_DIMS[-1] * HEADS[-1]
    edge_dim = EDGE_DIMS[-1] * HEADS[-1]
    node_vec = _set2set(x, params['node_s2s'], node_dim, NODE_S2S_ITERS, NODE_S2S_LAYERS)
    edge_vec = _set2set(ef, params['edge_s2s'], edge_dim, EDGE_S2S_ITERS, EDGE_S2S_LAYERS)
    vec = jnp.concatenate([node_vec[0], edge_vec[0]], axis=-1)[None]
    return _gated_mlp_pallas(vec, params['out'])[0]


# speedup vs baseline: 10.2505x; 8.9978x over previous
import functools

import jax
import jax.numpy as jnp
from jax import lax
from jax.experimental import pallas as pl
from jax.experimental.pallas import tpu as pltpu
from jax.experimental.pallas import tpu_sc as plsc

ATTN_DIMS = (32, 32)
EDGE_DIMS = (16, 16)
HEADS = (4, 4)
NODE_S2S_ITERS = 3
NODE_S2S_LAYERS = 2
EDGE_S2S_ITERS = 1
EDGE_S2S_LAYERS = 2

_HI = jax.lax.Precision.HIGHEST
_MESH = plsc.VectorSubcoreMesh(core_axis_name="c", subcore_axis_name="s")
_NW = 32
_SC_PARAMS = pltpu.CompilerParams(use_tc_tiling_on_sc=False)


def _hdot(a, b):
    return jax.lax.dot(a, b, precision=_HI)


def _tables_body(emb_ref, wnij_ref, wnode_ref, tnij_ref, th_ref):
    e = emb_ref[...]
    tnij_ref[...] = _hdot(e, wnij_ref[...])
    th_ref[...] = _hdot(e, wnode_ref[...])


def _node_tables(emb, wnij, wnode):
    t = emb.shape[0]
    return pl.pallas_call(
        _tables_body,
        out_shape=(
            jax.ShapeDtypeStruct((t, wnij.shape[1]), jnp.float32),
            jax.ShapeDtypeStruct((t, wnode.shape[1]), jnp.float32),
        ),
    )(emb, wnij, wnode)


def _node_expand(tnij, th, nfeats):
    n = nfeats.shape[0]
    ch = 400
    nb = n // ch
    nbt = -(-nb // _NW)

    @functools.partial(
        pl.kernel,
        mesh=_MESH,
        out_type=(
            jax.ShapeDtypeStruct((n, 128), jnp.float32),
            jax.ShapeDtypeStruct((n, 128), jnp.float32),
        ),
        scratch_types=[
            pltpu.VMEM((ch,), jnp.int32),
            pltpu.VMEM((ch, 128), jnp.float32),
            pltpu.VMEM((ch, 128), jnp.float32),
        ],
        compiler_params=_SC_PARAMS,
    )
    def k(tnij_h, th_h, nf_h, onij_h, oh_h, idx_v, b1, b2):
        wid = lax.axis_index("s") * 2 + lax.axis_index("c")
        for i in range(nbt):
            b = i * _NW + wid

            @pl.when(b < nb)
            def _():
                base = b * ch
                pltpu.sync_copy(nf_h.at[pl.ds(base, ch)], idx_v)
                pltpu.sync_copy(tnij_h.at[idx_v], b1)
                pltpu.sync_copy(th_h.at[idx_v], b2)
                pltpu.sync_copy(b1, onij_h.at[pl.ds(base, ch)])
                pltpu.sync_copy(b2, oh_h.at[pl.ds(base, ch)])

    return k(tnij, th, nfeats)


def _edge_gather(fnij, h, src, dst):
    e = src.shape[0]
    ch = 320
    nb = e // ch
    nbt = -(-nb // _NW)

    @functools.partial(
        pl.kernel,
        mesh=_MESH,
        out_type=(
            jax.ShapeDtypeStruct((e, 128), jnp.float32),
            jax.ShapeDtypeStruct((e, 128), jnp.float32),
            jax.ShapeDtypeStruct((e, 128), jnp.float32),
        ),
        scratch_types=[
            pltpu.VMEM((ch,), jnp.int32),
            pltpu.VMEM((ch,), jnp.int32),
            pltpu.VMEM((ch, 128), jnp.float32),
            pltpu.VMEM((ch, 128), jnp.float32),
            pltpu.VMEM((ch, 128), jnp.float32),
        ],
        compiler_params=_SC_PARAMS,
    )
    def k(fnij_h, h_h, src_h, dst_h, gs_h, gd_h, hs_h, sidx, didx, ba, bb, bc):
        wid = lax.axis_index("s") * 2 + lax.axis_index("c")
        for i in range(nbt):
            b = i * _NW + wid

            @pl.when(b < nb)
            def _():
                base = b * ch
                pltpu.sync_copy(src_h.at[pl.ds(base, ch)], sidx)
                pltpu.sync_copy(dst_h.at[pl.ds(base, ch)], didx)
                pltpu.sync_copy(fnij_h.at[sidx], ba)
                pltpu.sync_copy(fnij_h.at[didx], bb)
                pltpu.sync_copy(h_h.at[sidx], bc)
                pltpu.sync_copy(ba, gs_h.at[pl.ds(base, ch)])
                pltpu.sync_copy(bb, gd_h.at[pl.ds(base, ch)])
                pltpu.sync_copy(bc, hs_h.at[pl.ds(base, ch)])

    return k(fnij, h, src, dst)


_RANGES = 8
_RSIZE = 6250
_NROWS = 6272
_RPS = _NROWS // 16


def _segment_accumulate(m, eep, idx_flat):
    e = m.shape[0]
    ch = 400
    eps = e // 16
    nchunk = eps // ch
    rpc = _RANGES // 2

    z128 = jnp.zeros((_NROWS, 128), jnp.float32)
    z8 = jnp.zeros((_NROWS, 8), jnp.float32)

    @functools.partial(
        pl.kernel,
        mesh=_MESH,
        out_type=(
            jax.ShapeDtypeStruct((_RANGES * _NROWS, 128), jnp.float32),
            jax.ShapeDtypeStruct((_RANGES * _NROWS, 8), jnp.float32),
        ),
        scratch_types=[
            pltpu.VMEM_SHARED((_NROWS, 128), jnp.float32),
            pltpu.VMEM_SHARED((_NROWS, 8), jnp.float32),
            pltpu.VMEM((ch,), jnp.int32),
            pltpu.VMEM((ch, 128), jnp.float32),
            pltpu.VMEM((ch, 8), jnp.float32),
        ],
        compiler_params=_SC_PARAMS,
    )
    def k(m_h, e_h, idx_h, z128_h, z8_h, onum_h, oden_h,
          num_sh, den_sh, idx_v, m_v, e_v):
        c = lax.axis_index("c")
        s = lax.axis_index("s")
        rb = s * _RPS
        for r in range(rpc):
            g = c * rpc + r
            pltpu.sync_copy(z128_h.at[pl.ds(rb, _RPS)], num_sh.at[pl.ds(rb, _RPS)])
            pltpu.sync_copy(z8_h.at[pl.ds(rb, _RPS)], den_sh.at[pl.ds(rb, _RPS)])
            plsc.subcore_barrier()

            @pl.loop(0, nchunk)
            def _(i):
                base = s * eps + i * ch
                pltpu.sync_copy(idx_h.at[pl.ds(g * e + base, ch)], idx_v)
                pltpu.sync_copy(m_h.at[pl.ds(base, ch)], m_v)
                pltpu.sync_copy(e_h.at[pl.ds(base, ch)], e_v)
                pltpu.sync_copy(m_v, num_sh.at[idx_v], add=True)
                pltpu.sync_copy(e_v, den_sh.at[idx_v], add=True)

            plsc.subcore_barrier()
            ob = g * _NROWS + rb
            pltpu.sync_copy(num_sh.at[pl.ds(rb, _RPS)], onum_h.at[pl.ds(ob, _RPS)])
            pltpu.sync_copy(den_sh.at[pl.ds(rb, _RPS)], oden_h.at[pl.ds(ob, _RPS)])
            plsc.subcore_barrier()

    return k(m, eep, idx_flat, z128, z8)


def _range_indices(dst):
    base = (jnp.arange(_RANGES, dtype=jnp.int32) * _RSIZE)[:, None]
    loc = dst[None, :].astype(jnp.int32) - base
    return jnp.where((loc >= 0) & (loc < _RSIZE), loc, _RSIZE).reshape(-1)


def _lstm_forward(x, hs, cs, lstm_ps):
    new_hs, new_cs = [], []
    inp = x
    for lp, h, c in zip(lstm_ps, hs, cs):
        gates = inp @ lp['Wih'].T + h @ lp['Whh'].T + lp['bih'] + lp['bhh']
        i, f, g, o = jnp.split(gates, 4, axis=-1)
        c = jax.nn.sigmoid(f) * c + jax.nn.sigmoid(i) * jnp.tanh(g)
        h = jax.nn.sigmoid(o) * jnp.tanh(c)
        new_hs.append(h)
        new_cs.append(c)
        inp = h
    return inp, new_hs, new_cs


def _set2set(feat, lstm_ps, dim, n_iters, n_layers):
    q_star = jnp.zeros((1, 2 * dim), dtype=feat.dtype)
    hs = [jnp.zeros((1, dim), feat.dtype) for _ in range(n_layers)]
    cs = [jnp.zeros((1, dim), feat.dtype) for _ in range(n_layers)]
    for _ in range(n_iters):
        q, hs, cs = _lstm_forward(q_star, hs, cs, lstm_ps)
        e = feat @ q[0]
        alpha = jax.nn.softmax(e)
        readout = (feat * alpha[:, None]).sum(0, keepdims=True)
        q_star = jnp.concatenate([q, readout], axis=-1)
    return q_star


def _head_kernel(vec_ref, w1_ref, b1_ref, w2_ref, b2_ref,
                 gw1_ref, gb1_ref, gw2_ref, gb2_ref, out_ref):
    x = vec_ref[...]
    h = jax.nn.silu(_hdot(x, w1_ref[...]) + b1_ref[...])
    h = _hdot(h, w2_ref[...]) + b2_ref[...]
    g = jax.nn.silu(_hdot(x, gw1_ref[...]) + gb1_ref[...])
    g = jax.nn.sigmoid(_hdot(g, gw2_ref[...]) + gb2_ref[...])
    out_ref[...] = h * g


def _gated_mlp_pallas(vec, p):
    return pl.pallas_call(
        _head_kernel,
        out_shape=jax.ShapeDtypeStruct((1, 1), jnp.float32),
    )(vec, p['mlp_W1'], p['mlp_b1'][None], p['mlp_W2'], p['mlp_b2'][None],
      p['gate_W1'], p['gate_b1'][None], p['gate_W2'], p['gate_b2'][None])


def kernel(nfeats, efeats, edge_index, params):
    src, dst = edge_index[0], edge_index[1]
    n_nodes = nfeats.shape[0]
    n_edges = efeats.shape[0]
    ef = efeats
    idx_flat = _range_indices(dst)

    for li, (lp, out_n, out_e, H) in enumerate(
            zip(params['egat'], ATTN_DIMS, EDGE_DIMS, HEADS)):
        wnij = jnp.concatenate([lp['Wni'], lp['Wnj']], axis=1)
        if li == 0:
            tnij, th = _node_tables(params['emb'], wnij, lp['Wnode'])
            fnij, h = _node_expand(tnij, th, nfeats)
        else:
            fnij = _hdot(x, wnij)
            h = _hdot(x, lp['Wnode'])
        fsrc, fdst, hsrc = _edge_gather(fnij, h, src, dst)
        g = fsrc[:, :H * out_e] + fdst[:, H * out_e:]
        f = jax.nn.leaky_relu(g + ef @ lp['Wfij'], 0.2)
        fr = f.reshape(-1, H, out_e)
        z = (fr * lp['attn'][None]).sum(-1)
        ee = jnp.exp(z)
        hr = hsrc.reshape(-1, H, out_n)
        m = (hr * ee[:, :, None]).reshape(n_edges, -1)
        eep = jnp.pad(ee, ((0, 0), (0, 8 - H)))
        onum, oden = _segment_accumulate(m, eep, idx_flat)
        num = onum.reshape(_RANGES, _NROWS, 128)[:, :_RSIZE].reshape(n_nodes, H, out_n)
        den = oden.reshape(_RANGES, _NROWS, 8)[:, :_RSIZE, :H].reshape(n_nodes, H)
        h_out = num / (den[:, :, None] + 1e-9)
        x = jax.nn.elu(h_out.reshape(n_nodes, -1))
        ef = jax.nn.elu(f.reshape(n_edges, -1))

    node_dim = ATTN_DIMS[-1] * HEADS[-1]
    edge_dim = EDGE_DIMS[-1] * HEADS[-1]
    node_vec = _set2set(x, params['node_s2s'], node_dim, NODE_S2S_ITERS, NODE_S2S_LAYERS)
    edge_vec = _set2set(ef, params['edge_s2s'], edge_dim, EDGE_S2S_ITERS, EDGE_S2S_LAYERS)
    vec = jnp.concatenate([node_vec[0], edge_vec[0]], axis=-1)[None]
    return _gated_mlp_pallas(vec, params['out'])[0]


# speedup vs baseline: 10.2717x; 1.0021x over previous
import functools

import jax
import jax.numpy as jnp
from jax import lax
from jax.experimental import pallas as pl
from jax.experimental.pallas import tpu as pltpu
from jax.experimental.pallas import tpu_sc as plsc

ATTN_DIMS = (32, 32)
EDGE_DIMS = (16, 16)
HEADS = (4, 4)
NODE_S2S_ITERS = 3
NODE_S2S_LAYERS = 2
EDGE_S2S_ITERS = 1
EDGE_S2S_LAYERS = 2

_HI = jax.lax.Precision.HIGHEST
_MESH = plsc.VectorSubcoreMesh(core_axis_name="c", subcore_axis_name="s")
_NW = 32
_SC_PARAMS = pltpu.CompilerParams(use_tc_tiling_on_sc=False)


def _hdot(a, b):
    return jax.lax.dot(a, b, precision=_HI)


def _tables_body(emb_ref, wnij_ref, wnode_ref, tnij_ref, th_ref):
    e = emb_ref[...]
    tnij_ref[...] = _hdot(e, wnij_ref[...])
    th_ref[...] = _hdot(e, wnode_ref[...])


def _node_tables(emb, wnij, wnode):
    t = emb.shape[0]
    return pl.pallas_call(
        _tables_body,
        out_shape=(
            jax.ShapeDtypeStruct((t, wnij.shape[1]), jnp.float32),
            jax.ShapeDtypeStruct((t, wnode.shape[1]), jnp.float32),
        ),
    )(emb, wnij, wnode)


def _node_expand(tnij, th, nfeats):
    n = nfeats.shape[0]
    ch = 400
    nb = n // ch
    nbt = -(-nb // _NW)

    @functools.partial(
        pl.kernel,
        mesh=_MESH,
        out_type=(
            jax.ShapeDtypeStruct((n, 128), jnp.float32),
            jax.ShapeDtypeStruct((n, 128), jnp.float32),
        ),
        scratch_types=[
            pltpu.VMEM((ch,), jnp.int32),
            pltpu.VMEM((ch, 128), jnp.float32),
            pltpu.VMEM((ch, 128), jnp.float32),
        ],
        compiler_params=_SC_PARAMS,
    )
    def k(tnij_h, th_h, nf_h, onij_h, oh_h, idx_v, b1, b2):
        wid = lax.axis_index("s") * 2 + lax.axis_index("c")
        for i in range(nbt):
            b = i * _NW + wid

            @pl.when(b < nb)
            def _():
                base = b * ch
                pltpu.sync_copy(nf_h.at[pl.ds(base, ch)], idx_v)
                pltpu.sync_copy(tnij_h.at[idx_v], b1)
                pltpu.sync_copy(th_h.at[idx_v], b2)
                pltpu.sync_copy(b1, onij_h.at[pl.ds(base, ch)])
                pltpu.sync_copy(b2, oh_h.at[pl.ds(base, ch)])

    return k(tnij, th, nfeats)


def _edge_gather(fnij, h, src, dst):
    e = src.shape[0]
    ch = 320
    nb = e // ch
    nbt = -(-nb // _NW)

    @functools.partial(
        pl.kernel,
        mesh=_MESH,
        out_type=(
            jax.ShapeDtypeStruct((e, 128), jnp.float32),
            jax.ShapeDtypeStruct((e, 128), jnp.float32),
            jax.ShapeDtypeStruct((e, 128), jnp.float32),
        ),
        scratch_types=[
            pltpu.VMEM((ch,), jnp.int32),
            pltpu.VMEM((ch,), jnp.int32),
            pltpu.VMEM((ch, 128), jnp.float32),
            pltpu.VMEM((ch, 128), jnp.float32),
            pltpu.VMEM((ch, 128), jnp.float32),
        ],
        compiler_params=_SC_PARAMS,
    )
    def k(fnij_h, h_h, src_h, dst_h, gs_h, gd_h, hs_h, sidx, didx, ba, bb, bc):
        wid = lax.axis_index("s") * 2 + lax.axis_index("c")
        for i in range(nbt):
            b = i * _NW + wid

            @pl.when(b < nb)
            def _():
                base = b * ch
                pltpu.sync_copy(src_h.at[pl.ds(base, ch)], sidx)
                pltpu.sync_copy(dst_h.at[pl.ds(base, ch)], didx)
                pltpu.sync_copy(fnij_h.at[sidx], ba)
                pltpu.sync_copy(fnij_h.at[didx], bb)
                pltpu.sync_copy(h_h.at[sidx], bc)
                pltpu.sync_copy(ba, gs_h.at[pl.ds(base, ch)])
                pltpu.sync_copy(bb, gd_h.at[pl.ds(base, ch)])
                pltpu.sync_copy(bc, hs_h.at[pl.ds(base, ch)])

    return k(fnij, h, src, dst)


_RANGES = 8
_RSIZE = 6250
_NROWS = 6272
_RPS = _NROWS // 16


def _segment_accumulate(m, eep, idx_flat):
    e = m.shape[0]
    ch = 200
    eps = e // 16
    nchunk = eps // ch
    rpc = _RANGES // 2
    nbuf = 2

    z128 = jnp.zeros((_NROWS, 128), jnp.float32)
    z8 = jnp.zeros((_NROWS, 8), jnp.float32)

    @functools.partial(
        pl.kernel,
        mesh=_MESH,
        out_type=(
            jax.ShapeDtypeStruct((_RANGES * _NROWS, 128), jnp.float32),
            jax.ShapeDtypeStruct((_RANGES * _NROWS, 8), jnp.float32),
        ),
        scratch_types=[
            pltpu.VMEM_SHARED((_NROWS, 128), jnp.float32),
            pltpu.VMEM_SHARED((_NROWS, 8), jnp.float32),
            pltpu.VMEM((nbuf, ch), jnp.int32),
            pltpu.VMEM((nbuf, ch, 128), jnp.float32),
            pltpu.VMEM((nbuf, ch, 8), jnp.float32),
            pltpu.SemaphoreType.DMA,
            pltpu.SemaphoreType.DMA,
        ],
        compiler_params=_SC_PARAMS,
    )
    def k(m_h, e_h, idx_h, z128_h, z8_h, onum_h, oden_h,
          num_sh, den_sh, idx_v, m_v, e_v, sem0, sem1):
        c = lax.axis_index("c")
        s = lax.axis_index("s")
        rb = s * _RPS
        sems = (sem0, sem1)

        def start(g, i, b):
            base = s * eps + i * ch
            pltpu.async_copy(idx_h.at[pl.ds(g * e + base, ch)], idx_v.at[b], sems[b])
            pltpu.async_copy(m_h.at[pl.ds(base, ch)], m_v.at[b], sems[b])
            pltpu.async_copy(e_h.at[pl.ds(base, ch)], e_v.at[b], sems[b])

        def drain(g, i, b):
            base = s * eps + i * ch
            pltpu.make_async_copy(
                idx_h.at[pl.ds(g * e + base, ch)], idx_v.at[b], sems[b]).wait()
            pltpu.make_async_copy(
                m_h.at[pl.ds(base, ch)], m_v.at[b], sems[b]).wait()
            pltpu.make_async_copy(
                e_h.at[pl.ds(base, ch)], e_v.at[b], sems[b]).wait()

        for r in range(rpc):
            g = c * rpc + r
            pltpu.sync_copy(z128_h.at[pl.ds(rb, _RPS)], num_sh.at[pl.ds(rb, _RPS)])
            pltpu.sync_copy(z8_h.at[pl.ds(rb, _RPS)], den_sh.at[pl.ds(rb, _RPS)])
            for b in range(nbuf):
                start(g, b, b)
            plsc.subcore_barrier()

            @pl.loop(0, nchunk, step=nbuf)
            def _(i):
                for b in range(nbuf):
                    drain(g, i + b, b)
                    pltpu.sync_copy(m_v.at[b], num_sh.at[idx_v.at[b]], add=True)
                    pltpu.sync_copy(e_v.at[b], den_sh.at[idx_v.at[b]], add=True)

                    @pl.when(i + b + nbuf < nchunk)
                    def _():
                        start(g, i + b + nbuf, b)

            plsc.subcore_barrier()
            ob = g * _NROWS + rb
            pltpu.sync_copy(num_sh.at[pl.ds(rb, _RPS)], onum_h.at[pl.ds(ob, _RPS)])
            pltpu.sync_copy(den_sh.at[pl.ds(rb, _RPS)], oden_h.at[pl.ds(ob, _RPS)])
            plsc.subcore_barrier()

    return k(m, eep, idx_flat, z128, z8)


def _range_indices(dst):
    base = (jnp.arange(_RANGES, dtype=jnp.int32) * _RSIZE)[:, None]
    loc = dst[None, :].astype(jnp.int32) - base
    return jnp.where((loc >= 0) & (loc < _RSIZE), loc, _RSIZE).reshape(-1)


def _lstm_forward(x, hs, cs, lstm_ps):
    new_hs, new_cs = [], []
    inp = x
    for lp, h, c in zip(lstm_ps, hs, cs):
        gates = inp @ lp['Wih'].T + h @ lp['Whh'].T + lp['bih'] + lp['bhh']
        i, f, g, o = jnp.split(gates, 4, axis=-1)
        c = jax.nn.sigmoid(f) * c + jax.nn.sigmoid(i) * jnp.tanh(g)
        h = jax.nn.sigmoid(o) * jnp.tanh(c)
        new_hs.append(h)
        new_cs.append(c)
        inp = h
    return inp, new_hs, new_cs


def _set2set(feat, lstm_ps, dim, n_iters, n_layers):
    q_star = jnp.zeros((1, 2 * dim), dtype=feat.dtype)
    hs = [jnp.zeros((1, dim), feat.dtype) for _ in range(n_layers)]
    cs = [jnp.zeros((1, dim), feat.dtype) for _ in range(n_layers)]
    for _ in range(n_iters):
        q, hs, cs = _lstm_forward(q_star, hs, cs, lstm_ps)
        e = feat @ q[0]
        alpha = jax.nn.softmax(e)
        readout = (feat * alpha[:, None]).sum(0, keepdims=True)
        q_star = jnp.concatenate([q, readout], axis=-1)
    return q_star


def _head_kernel(vec_ref, w1_ref, b1_ref, w2_ref, b2_ref,
                 gw1_ref, gb1_ref, gw2_ref, gb2_ref, out_ref):
    x = vec_ref[...]
    h = jax.nn.silu(_hdot(x, w1_ref[...]) + b1_ref[...])
    h = _hdot(h, w2_ref[...]) + b2_ref[...]
    g = jax.nn.silu(_hdot(x, gw1_ref[...]) + gb1_ref[...])
    g = jax.nn.sigmoid(_hdot(g, gw2_ref[...]) + gb2_ref[...])
    out_ref[...] = h * g


def _gated_mlp_pallas(vec, p):
    return pl.pallas_call(
        _head_kernel,
        out_shape=jax.ShapeDtypeStruct((1, 1), jnp.float32),
    )(vec, p['mlp_W1'], p['mlp_b1'][None], p['mlp_W2'], p['mlp_b2'][None],
      p['gate_W1'], p['gate_b1'][None], p['gate_W2'], p['gate_b2'][None])


def kernel(nfeats, efeats, edge_index, params):
    src, dst = edge_index[0], edge_index[1]
    n_nodes = nfeats.shape[0]
    n_edges = efeats.shape[0]
    ef = efeats
    idx_flat = _range_indices(dst)

    for li, (lp, out_n, out_e, H) in enumerate(
            zip(params['egat'], ATTN_DIMS, EDGE_DIMS, HEADS)):
        wnij = jnp.concatenate([lp['Wni'], lp['Wnj']], axis=1)
        if li == 0:
            tnij, th = _node_tables(params['emb'], wnij, lp['Wnode'])
            fnij, h = _node_expand(tnij, th, nfeats)
        else:
            fnij = _hdot(x, wnij)
            h = _hdot(x, lp['Wnode'])
        fsrc, fdst, hsrc = _edge_gather(fnij, h, src, dst)
        g = fsrc[:, :H * out_e] + fdst[:, H * out_e:]
        f = jax.nn.leaky_relu(g + ef @ lp['Wfij'], 0.2)
        fr = f.reshape(-1, H, out_e)
        z = (fr * lp['attn'][None]).sum(-1)
        ee = jnp.exp(z)
        hr = hsrc.reshape(-1, H, out_n)
        m = (hr * ee[:, :, None]).reshape(n_edges, -1)
        eep = jnp.pad(ee, ((0, 0), (0, 8 - H)))
        onum, oden = _segment_accumulate(m, eep, idx_flat)
        num = onum.reshape(_RANGES, _NROWS, 128)[:, :_RSIZE].reshape(n_nodes, H, out_n)
        den = oden.reshape(_RANGES, _NROWS, 8)[:, :_RSIZE, :H].reshape(n_nodes, H)
        h_out = num / (den[:, :, None] + 1e-9)
        x = jax.nn.elu(h_out.reshape(n_nodes, -1))
        ef = jax.nn.elu(f.reshape(n_edges, -1))

    node_dim = ATTN_DIMS[-1] * HEADS[-1]
    edge_dim = EDGE_DIMS[-1] * HEADS[-1]
    node_vec = _set2set(x, params['node_s2s'], node_dim, NODE_S2S_ITERS, NODE_S2S_LAYERS)
    edge_vec = _set2set(ef, params['edge_s2s'], edge_dim, EDGE_S2S_ITERS, EDGE_S2S_LAYERS)
    vec = jnp.concatenate([node_vec[0], edge_vec[0]], axis=-1)[None]
    return _gated_mlp_pallas(vec, params['out'])[0]
